# Initial kernel scaffold; baseline (speedup 1.0000x reference)
import functools

import jax
import jax.numpy as jnp
from jax import lax
from jax.experimental import pallas as pl
from jax.experimental.pallas import tpu as pltpu
from jax.experimental.pallas import tpu_sc as plsc

_B, _N = 8, 2048
_L = 16
_NC, _NS = 2, 16
_NW = _NC * _NS
_RPW = _B * _N // _NW
_WPB = _NW // _B
_ROWS = 2
_CHUNKS = _N // _L
_BIG = jnp.float32(3.0e38)

_RADIUS = 0.07
_H = 0.03
_EPS = 1e-12


def _sc_body(pred_hbm, gt_hbm, out_min, out_top, pco, gco, psc, gsc,
             minbuf, topbuf):
    wid = lax.axis_index("s") * _NC + lax.axis_index("c")
    b = wid // _WPB
    base = (wid % _WPB) * _RPW

    pltpu.sync_copy(pred_hbm.at[b], pco)
    pltpu.sync_copy(gt_hbm.at[b], gco)

    def prep(c, carry):
        s = pl.ds(c * _L, _L)
        x = pco[0, s]
        y = pco[1, s]
        z = pco[2, s]
        psc[0, s] = -2.0 * x
        psc[1, s] = -2.0 * y
        psc[2, s] = -2.0 * z
        psc[3, s] = x * x + y * y + z * z
        x = gco[0, s]
        y = gco[1, s]
        z = gco[2, s]
        gsc[0, s] = -2.0 * x
        gsc[1, s] = -2.0 * y
        gsc[2, s] = -2.0 * z
        gsc[3, s] = x * x + y * y + z * z
        return carry

    lax.fori_loop(0, _CHUNKS, prep, 0)

    zero16 = jnp.zeros((_L,), jnp.int32)
    one16 = jnp.full((_L,), 1, jnp.int32)
    two16 = jnp.full((_L,), 2, jnp.int32)

    def rowgrp(g, carry):
        px, py, pz, pn = [], [], [], []
        for i in range(_ROWS):
            ni = jnp.full((_L,), base + g * _ROWS + i, jnp.int32)
            x = plsc.load_gather(pco, [zero16, ni])
            y = plsc.load_gather(pco, [one16, ni])
            z = plsc.load_gather(pco, [two16, ni])
            px.append(x)
            py.append(y)
            pz.append(z)
            pn.append(x * x + y * y + z * z)

        init = tuple(jnp.full((_L,), _BIG) for _ in range(_ROWS * 6))

        def chunk(c, st):
            st = list(st)
            s = pl.ds(c * _L, _L)
            g0 = gsc[0, s]
            g1 = gsc[1, s]
            g2 = gsc[2, s]
            g3 = gsc[3, s]
            p0 = psc[0, s]
            p1 = psc[1, s]
            p2 = psc[2, s]
            p3 = psc[3, s]
            out = []
            for i in range(_ROWS):
                m, t0, t1, t2, t3, t4 = st[i * 6:(i + 1) * 6]
                d = px[i] * g0 + py[i] * g1 + pz[i] * g2 + (g3 + pn[i])
                m = jnp.minimum(m, d)
                e = px[i] * p0 + py[i] * p1 + pz[i] * p2 + (p3 + pn[i])
                lo = jnp.minimum(t0, e)
                e = jnp.maximum(t0, e)
                t0 = lo
                lo = jnp.minimum(t1, e)
                e = jnp.maximum(t1, e)
                t1 = lo
                lo = jnp.minimum(t2, e)
                e = jnp.maximum(t2, e)
                t2 = lo
                lo = jnp.minimum(t3, e)
                e = jnp.maximum(t3, e)
                t3 = lo
                t4 = jnp.minimum(t4, e)
                out += [m, t0, t1, t2, t3, t4]
            return tuple(out)

        res = lax.fori_loop(0, _CHUNKS, chunk, init)
        for i in range(_ROWS):
            r = g * _ROWS + i
            vals = res[i * 6:(i + 1) * 6]
            minbuf[r, :] = vals[0]
            for k in range(5):
                topbuf[r, k, :] = vals[1 + k]
        return carry

    lax.fori_loop(0, _RPW // _ROWS, rowgrp, 0)

    pltpu.sync_copy(minbuf, out_min.at[wid])
    pltpu.sync_copy(topbuf, out_top.at[wid])


_sc_knn = functools.partial(
    pl.kernel,
    out_type=(
        jax.ShapeDtypeStruct((_NW, _RPW, _L), jnp.float32),
        jax.ShapeDtypeStruct((_NW, _RPW, 5, _L), jnp.float32),
    ),
    mesh=plsc.VectorSubcoreMesh(core_axis_name="c", subcore_axis_name="s"),
    scratch_types=[
        pltpu.VMEM((3, _N), jnp.float32),
        pltpu.VMEM((3, _N), jnp.float32),
        pltpu.VMEM((4, _N), jnp.float32),
        pltpu.VMEM((4, _N), jnp.float32),
        pltpu.VMEM((_RPW, _L), jnp.float32),
        pltpu.VMEM((_RPW, 5, _L), jnp.float32),
    ],
)(_sc_body)


def _reduce_body(mins_ref, top_ref, rad_ref, emd_ref, uni_ref):
    mins = mins_ref[...]
    rowmin = jnp.min(mins, axis=0)
    per_b = jnp.sum(rowmin, axis=1, keepdims=True)
    scaled = per_b / rad_ref[:, :1]
    emd_ref[0, 0] = jnp.sum(scaled) * (250.0 / (3.0 * _N * _B))

    data = top_ref[...]
    m = jnp.min(data, axis=1, keepdims=True)
    data = jnp.where(data == m, _BIG, data)
    acc = jnp.zeros((), jnp.float32)
    for k in range(4):
        m = jnp.min(data, axis=1, keepdims=True)
        v = jnp.maximum(m, _EPS)
        dist = jnp.sqrt(v)
        w = jnp.exp(v * (-1.0 / (_H * _H)))
        acc = acc + jnp.sum((_RADIUS - dist) * w)
        if k < 3:
            data = jnp.where(data == m, _BIG, data)
    uni_ref[0, 0] = acc / (_B * _N * 4.0)


def kernel(pred_fullpoint, gt_fullpoint, radius_data):
    pred_t = jnp.transpose(pred_fullpoint, (0, 2, 1))
    gt_t = jnp.transpose(gt_fullpoint, (0, 2, 1))

    mins, tops = _sc_knn(pred_t, gt_t)

    mins_t = jnp.transpose(mins.reshape(_B, _N, _L), (2, 0, 1))
    top_flat = tops.reshape(_B * _N, 5 * _L)
    top_pad = jnp.concatenate(
        [top_flat, jnp.full((_B * _N, 128 - 5 * _L), _BIG, jnp.float32)],
        axis=1)
    rad = jnp.broadcast_to(radius_data.reshape(_B, 1), (_B, 128))

    emd, uni = pl.pallas_call(
        _reduce_body,
        out_shape=(
            jax.ShapeDtypeStruct((1, 1), jnp.float32),
            jax.ShapeDtypeStruct((1, 1), jnp.float32),
        ),
        out_specs=(
            pl.BlockSpec(memory_space=pltpu.SMEM),
            pl.BlockSpec(memory_space=pltpu.SMEM),
        ),
    )(mins_t, top_pad, rad)

    return (emd[0, 0], uni[0, 0])


# speedup vs baseline: 7.2729x; 7.2729x over previous
import functools

import jax
import jax.numpy as jnp
from jax import lax
from jax.experimental import pallas as pl
from jax.experimental.pallas import tpu as pltpu
from jax.experimental.pallas import tpu_sc as plsc

_B, _N = 8, 2048
_L = 16
_NC, _NS = 2, 16
_NW = _NC * _NS
_RPW = _B * _N // _NW
_WPB = _NW // _B
_GRP = _RPW // _L
_CHUNKS = _N // _L
_BIG = float(3.0e38)

_RADIUS = 0.07
_H = 0.03
_EPS = 1e-12


def _sc_body(pred_hbm, gt_hbm, bpred_hbm, bgt_hbm, out_mk, out_mv,
             out_tk, out_tv, pco, gco, bpc, bgc, mkb, mvb, tkb, tvb):
    wid = lax.axis_index("s") * _NC + lax.axis_index("c")
    b = wid // _WPB
    base = (wid % _WPB) * _RPW

    pltpu.sync_copy(pred_hbm.at[b], pco)
    pltpu.sync_copy(gt_hbm.at[b], gco)
    pltpu.sync_copy(bpred_hbm.at[b], bpc)
    pltpu.sync_copy(bgt_hbm.at[b], bgc)

    def grp16(g, carry):
        rbase = base + g * _L
        pxc = pco[pl.ds(rbase, _L)]
        pyc = pco[pl.ds(_N + rbase, _L)]
        pzc = pco[pl.ds(2 * _N + rbase, _L)]
        bxc = bpc[pl.ds(rbase, _L)]
        byc = bpc[pl.ds(_N + rbase, _L)]
        bzc = bpc[pl.ds(2 * _N + rbase, _L)]

        for l in range(_L):
            px = jnp.broadcast_to(pxc[l], (_L,))
            py = jnp.broadcast_to(pyc[l], (_L,))
            pz = jnp.broadcast_to(pzc[l], (_L,))
            bx = jnp.broadcast_to(bxc[l], (_L,))
            by = jnp.broadcast_to(byc[l], (_L,))
            bz = jnp.broadcast_to(bzc[l], (_L,))
            pn = px * px + py * py + pz * pz

            init = (jnp.full((_L,), _BIG, jnp.float32),) * 12

            def chunk(c, st, px=px, py=py, pz=pz, bx=bx, by=by, bz=bz,
                      pn=pn):
                mk, mv = st[0], st[1]
                tk = list(st[2:7])
                tv = list(st[7:12])
                s0 = pl.ds(c * _L, _L)
                s1 = pl.ds(_N + c * _L, _L)
                s2 = pl.ds(2 * _N + c * _L, _L)

                gx = gco[s0]
                gy = gco[s1]
                gz = gco[s2]
                gn = gx * gx + gy * gy + gz * gz
                dot = bx * bgc[s0] + by * bgc[s1] + bz * bgc[s2]
                key = (pn + gn) - 2.0 * dot
                dx = px - gx
                dy = py - gy
                dz = pz - gz
                val = dx * dx + dy * dy + dz * dz
                cond = key < mk
                mk = jnp.minimum(mk, key)
                mv = jnp.where(cond, val, mv)

                qx = pco[s0]
                qy = pco[s1]
                qz = pco[s2]
                qn = qx * qx + qy * qy + qz * qz
                edot = bx * bpc[s0] + by * bpc[s1] + bz * bpc[s2]
                ekey = (pn + qn) - 2.0 * edot
                ex = px - qx
                ey = py - qy
                ez = pz - qz
                ev = ex * ex + ey * ey + ez * ez
                for lev in range(4):
                    cnd = ekey < tk[lev]
                    nk = jnp.minimum(tk[lev], ekey)
                    xk = jnp.maximum(tk[lev], ekey)
                    nv = jnp.where(cnd, ev, tv[lev])
                    xv = jnp.where(cnd, tv[lev], ev)
                    tk[lev] = nk
                    tv[lev] = nv
                    ekey = xk
                    ev = xv
                cnd = ekey < tk[4]
                tk[4] = jnp.minimum(tk[4], ekey)
                tv[4] = jnp.where(cnd, ev, tv[4])
                return (mk, mv) + tuple(tk) + tuple(tv)

            st = lax.fori_loop(0, _CHUNKS, chunk, init)
            r = g * _L + l
            mkb[pl.ds(r * _L, _L)] = st[0]
            mvb[pl.ds(r * _L, _L)] = st[1]
            for k in range(5):
                tkb[pl.ds((r * 5 + k) * _L, _L)] = st[2 + k]
                tvb[pl.ds((r * 5 + k) * _L, _L)] = st[7 + k]
        return carry

    lax.fori_loop(0, _GRP, grp16, 0)

    pltpu.sync_copy(mkb, out_mk.at[wid])
    pltpu.sync_copy(mvb, out_mv.at[wid])
    pltpu.sync_copy(tkb, out_tk.at[wid])
    pltpu.sync_copy(tvb, out_tv.at[wid])


_sc_knn = functools.partial(
    pl.kernel,
    out_type=(
        jax.ShapeDtypeStruct((_NW, _RPW * _L), jnp.float32),
        jax.ShapeDtypeStruct((_NW, _RPW * _L), jnp.float32),
        jax.ShapeDtypeStruct((_NW, _RPW * 5 * _L), jnp.float32),
        jax.ShapeDtypeStruct((_NW, _RPW * 5 * _L), jnp.float32),
    ),
    mesh=plsc.VectorSubcoreMesh(core_axis_name="c", subcore_axis_name="s",
                                num_cores=_NC, num_subcores=_NS),
    scratch_types=[
        pltpu.VMEM((3 * _N,), jnp.float32),
        pltpu.VMEM((3 * _N,), jnp.float32),
        pltpu.VMEM((3 * _N,), jnp.float32),
        pltpu.VMEM((3 * _N,), jnp.float32),
        pltpu.VMEM((_RPW * _L,), jnp.float32),
        pltpu.VMEM((_RPW * _L,), jnp.float32),
        pltpu.VMEM((_RPW * 5 * _L,), jnp.float32),
        pltpu.VMEM((_RPW * 5 * _L,), jnp.float32),
    ],
)(_sc_body)


def _reduce_body(mk_ref, mv_ref, tk_ref, tv_ref, rad_ref, emd_ref, uni_ref):
    mk = mk_ref[...]
    mv = mv_ref[...]
    rowkey = jnp.min(mk, axis=0)
    rowval = jnp.min(jnp.where(mk == rowkey[None], mv, _BIG), axis=0)
    per_b = jnp.sum(rowval, axis=1, keepdims=True)
    scaled = per_b / rad_ref[:, :1]
    emd_ref[0, 0] = jnp.sum(scaled) * (250.0 / (3.0 * _N * _B))

    tk = tk_ref[...]
    tv = tv_ref[...]
    k0 = jnp.min(tk, axis=1, keepdims=True)
    tk = jnp.where(tk == k0, _BIG, tk)
    acc = jnp.zeros((), jnp.float32)
    for k in range(4):
        kk = jnp.min(tk, axis=1, keepdims=True)
        sel = tk == kk
        vv = jnp.min(jnp.where(sel, tv, _BIG), axis=1, keepdims=True)
        v = jnp.maximum(vv, _EPS)
        dist = jnp.sqrt(v)
        w = jnp.exp(v * (-1.0 / (_H * _H)))
        acc = acc + jnp.sum((_RADIUS - dist) * w)
        if k < 3:
            tk = jnp.where(sel, _BIG, tk)
    uni_ref[0, 0] = acc / (_B * _N * 4.0)


def kernel(pred_fullpoint, gt_fullpoint, radius_data):
    pred_f = jnp.transpose(pred_fullpoint, (0, 2, 1)).reshape(_B, 3 * _N)
    gt_f = jnp.transpose(gt_fullpoint, (0, 2, 1)).reshape(_B, 3 * _N)
    def round_bf16(x):
        u = lax.bitcast_convert_type(x, jnp.int32)
        r = (u + ((u >> 16) & 1) + 0x7FFF) & jnp.int32(-65536)
        return lax.bitcast_convert_type(r, jnp.float32)

    bpred_f = round_bf16(pred_f)
    bgt_f = round_bf16(gt_f)

    mk, mv, tk, tv = _sc_knn(pred_f, gt_f, bpred_f, bgt_f)

    mk_t = jnp.transpose(mk.reshape(_B, _N, _L), (2, 0, 1))
    mv_t = jnp.transpose(mv.reshape(_B, _N, _L), (2, 0, 1))
    pad = jnp.full((_B * _N, 128 - 5 * _L), _BIG, jnp.float32)
    tk_p = jnp.concatenate([tk.reshape(_B * _N, 5 * _L), pad], axis=1)
    tv_p = jnp.concatenate([tv.reshape(_B * _N, 5 * _L), pad], axis=1)
    rad = jnp.broadcast_to(radius_data.reshape(_B, 1), (_B, 128))

    emd, uni = pl.pallas_call(
        _reduce_body,
        out_shape=(
            jax.ShapeDtypeStruct((1, 1), jnp.float32),
            jax.ShapeDtypeStruct((1, 1), jnp.float32),
        ),
        out_specs=(
            pl.BlockSpec(memory_space=pltpu.SMEM),
            pl.BlockSpec(memory_space=pltpu.SMEM),
        ),
    )(mk_t, mv_t, tk_p, tv_p, rad)

    return (emd[0, 0], uni[0, 0])


# speedup vs baseline: 11.7846x; 1.6203x over previous
import functools

import jax
import jax.numpy as jnp
from jax import lax
from jax.experimental import pallas as pl
from jax.experimental.pallas import tpu as pltpu
from jax.experimental.pallas import tpu_sc as plsc

_B, _N = 8, 2048
_L = 16
_NC, _NS = 2, 16
_NW = _NC * _NS
_RPW = _B * _N // _NW
_WPB = _NW // _B
_GRP = _RPW // _L
_CHUNKS = _N // _L
_TR = 256
_NT = _N // _TR
_BIG = float(3.0e38)

_RADIUS = 0.07
_H = 0.03
_EPS = 1e-12


def _sc_body(pred_hbm, bpred_hbm, out_tk, out_tv, pco, bpc, qnb, tkb, tvb):
    wid = lax.axis_index("s") * _NC + lax.axis_index("c")
    b = wid // _WPB
    base = (wid % _WPB) * _RPW

    pltpu.sync_copy(pred_hbm.at[b], pco)
    pltpu.sync_copy(bpred_hbm.at[b], bpc)

    def prep(c, carry):
        s = pl.ds(c * _L, _L)
        qx = pco[s]
        qy = pco[pl.ds(_N + c * _L, _L)]
        qz = pco[pl.ds(2 * _N + c * _L, _L)]
        qnb[s] = qx * qx + qy * qy + qz * qz
        return carry

    lax.fori_loop(0, _CHUNKS, prep, 0)

    def grp16(g, carry):
        rbase = base + g * _L
        pxc = pco[pl.ds(rbase, _L)]
        pyc = pco[pl.ds(_N + rbase, _L)]
        pzc = pco[pl.ds(2 * _N + rbase, _L)]
        bxc = bpc[pl.ds(rbase, _L)]
        byc = bpc[pl.ds(_N + rbase, _L)]
        bzc = bpc[pl.ds(2 * _N + rbase, _L)]
        pnc = qnb[pl.ds(rbase, _L)]

        for l in range(_L):
            px = jnp.broadcast_to(pxc[l], (_L,))
            py = jnp.broadcast_to(pyc[l], (_L,))
            pz = jnp.broadcast_to(pzc[l], (_L,))
            bx = jnp.broadcast_to(bxc[l], (_L,))
            by = jnp.broadcast_to(byc[l], (_L,))
            bz = jnp.broadcast_to(bzc[l], (_L,))
            pn = jnp.broadcast_to(pnc[l], (_L,))

            init = (jnp.full((_L,), _BIG, jnp.float32),) * 10

            def chunk(c, st, px=px, py=py, pz=pz, bx=bx, by=by, bz=bz,
                      pn=pn):
                tk = list(st[0:5])
                tv = list(st[5:10])
                s0 = pl.ds(c * _L, _L)
                s1 = pl.ds(_N + c * _L, _L)
                s2 = pl.ds(2 * _N + c * _L, _L)

                qx = pco[s0]
                qy = pco[s1]
                qz = pco[s2]
                edot = bx * bpc[s0] + by * bpc[s1] + bz * bpc[s2]
                ekey = (pn + qnb[s0]) - 2.0 * edot
                ex = px - qx
                ey = py - qy
                ez = pz - qz
                ev = ex * ex + ey * ey + ez * ez
                for lev in range(4):
                    cnd = ekey < tk[lev]
                    nk = jnp.minimum(tk[lev], ekey)
                    xk = jnp.maximum(tk[lev], ekey)
                    nv = jnp.where(cnd, ev, tv[lev])
                    xv = jnp.where(cnd, tv[lev], ev)
                    tk[lev] = nk
                    tv[lev] = nv
                    ekey = xk
                    ev = xv
                cnd = ekey < tk[4]
                tk[4] = jnp.minimum(tk[4], ekey)
                tv[4] = jnp.where(cnd, ev, tv[4])
                return tuple(tk) + tuple(tv)

            st = lax.fori_loop(0, _CHUNKS, chunk, init, unroll=2)
            r = g * _L + l
            for k in range(5):
                tkb[pl.ds((r * 5 + k) * _L, _L)] = st[k]
                tvb[pl.ds((r * 5 + k) * _L, _L)] = st[5 + k]
        return carry

    lax.fori_loop(0, _GRP, grp16, 0)

    pltpu.sync_copy(tkb, out_tk.at[wid])
    pltpu.sync_copy(tvb, out_tv.at[wid])


_sc_knn = functools.partial(
    pl.kernel,
    out_type=(
        jax.ShapeDtypeStruct((_NW, _RPW * 5 * _L), jnp.float32),
        jax.ShapeDtypeStruct((_NW, _RPW * 5 * _L), jnp.float32),
    ),
    mesh=plsc.VectorSubcoreMesh(core_axis_name="c", subcore_axis_name="s",
                                num_cores=_NC, num_subcores=_NS),
    scratch_types=[
        pltpu.VMEM((3 * _N,), jnp.float32),
        pltpu.VMEM((3 * _N,), jnp.float32),
        pltpu.VMEM((_N,), jnp.float32),
        pltpu.VMEM((_RPW * 5 * _L,), jnp.float32),
        pltpu.VMEM((_RPW * 5 * _L,), jnp.float32),
    ],
)(_sc_body)


def _emd_body(p_ref, bp_ref, g_ref, bg_ref, out_ref):
    p = p_ref[0]
    bp = bp_ref[0]
    g = g_ref[0]
    bg = bg_ref[0]
    a2 = jnp.sum(p * p, axis=1, keepdims=True)
    b2 = jnp.sum(g * g, axis=0, keepdims=True)
    dot_k = jax.lax.dot_general(
        bp, bg, (((1,), (0,)), ((), ())),
        preferred_element_type=jnp.float32)
    key = (a2 + b2) - 2.0 * dot_k
    dot_v = jax.lax.dot_general(
        p, g, (((1,), (0,)), ((), ())),
        preferred_element_type=jnp.float32,
        precision=jax.lax.Precision.HIGHEST)
    val = (a2 + b2) - 2.0 * dot_v
    rowkey = jnp.min(key, axis=1, keepdims=True)
    rowval = jnp.min(jnp.where(key == rowkey, val, _BIG), axis=1,
                     keepdims=True)
    out_ref[pl.program_id(0), pl.program_id(1)] = jnp.sum(rowval)


def _reduce_body(pt_ref, tk_ref, tv_ref, rad_ref, emd_ref, uni_ref):
    tot = jnp.float32(0.0)
    for b in range(_B):
        pb = jnp.float32(0.0)
        for t in range(_NT):
            pb = pb + pt_ref[b, t]
        tot = tot + pb / rad_ref[b, 0]
    emd_ref[0, 0] = tot * (250.0 / (3.0 * _N * _B))

    tk = tk_ref[...]
    tv = tv_ref[...]
    k0 = jnp.min(tk, axis=1, keepdims=True)
    tk = jnp.where(tk == k0, _BIG, tk)
    acc = jnp.zeros((), jnp.float32)
    for k in range(4):
        kk = jnp.min(tk, axis=1, keepdims=True)
        sel = tk == kk
        vv = jnp.min(jnp.where(sel, tv, _BIG), axis=1, keepdims=True)
        v = jnp.maximum(vv, _EPS)
        dist = jnp.sqrt(v)
        w = jnp.exp(v * (-1.0 / (_H * _H)))
        acc = acc + jnp.sum((_RADIUS - dist) * w)
        if k < 3:
            tk = jnp.where(sel, _BIG, tk)
    uni_ref[0, 0] = acc / (_B * _N * 4.0)


def _round_bf16(x):
    u = lax.bitcast_convert_type(x, jnp.int32)
    r = (u + ((u >> 16) & 1) + 0x7FFF) & jnp.int32(-65536)
    return lax.bitcast_convert_type(r, jnp.float32)


def kernel(pred_fullpoint, gt_fullpoint, radius_data):
    pred_t = jnp.transpose(pred_fullpoint, (0, 2, 1))
    pred_f = pred_t.reshape(_B, 3 * _N)
    bpred_f = _round_bf16(pred_f)

    zpad_r = jnp.zeros((_B, _N, 5), jnp.float32)
    p_pad = jnp.concatenate([pred_fullpoint, zpad_r], axis=2)
    bp_pad = _round_bf16(p_pad)
    zpad_c = jnp.zeros((_B, 5, _N), jnp.float32)
    g_pad = jnp.concatenate(
        [jnp.transpose(gt_fullpoint, (0, 2, 1)), zpad_c], axis=1)
    bg_pad = _round_bf16(g_pad)

    tk, tv = _sc_knn(pred_f, bpred_f)

    partials = pl.pallas_call(
        _emd_body,
        grid=(_B, _NT),
        in_specs=[
            pl.BlockSpec((1, _TR, 8), lambda b, t: (b, t, 0)),
            pl.BlockSpec((1, _TR, 8), lambda b, t: (b, t, 0)),
            pl.BlockSpec((1, 8, _N), lambda b, t: (b, 0, 0)),
            pl.BlockSpec((1, 8, _N), lambda b, t: (b, 0, 0)),
        ],
        out_specs=pl.BlockSpec((_B, _NT), lambda b, t: (0, 0),
                               memory_space=pltpu.SMEM),
        out_shape=jax.ShapeDtypeStruct((_B, _NT), jnp.float32),
    )(p_pad, bp_pad, g_pad, bg_pad)
    pad = jnp.full((_B * _N, 128 - 5 * _L), _BIG, jnp.float32)
    tk_p = jnp.concatenate([tk.reshape(_B * _N, 5 * _L), pad], axis=1)
    tv_p = jnp.concatenate([tv.reshape(_B * _N, 5 * _L), pad], axis=1)
    rad = radius_data.reshape(_B, 1)

    emd, uni = pl.pallas_call(
        _reduce_body,
        in_specs=[
            pl.BlockSpec(memory_space=pltpu.SMEM),
            pl.BlockSpec((_B * _N, 128), lambda: (0, 0)),
            pl.BlockSpec((_B * _N, 128), lambda: (0, 0)),
            pl.BlockSpec(memory_space=pltpu.SMEM),
        ],
        out_shape=(
            jax.ShapeDtypeStruct((1, 1), jnp.float32),
            jax.ShapeDtypeStruct((1, 1), jnp.float32),
        ),
        out_specs=(
            pl.BlockSpec(memory_space=pltpu.SMEM),
            pl.BlockSpec(memory_space=pltpu.SMEM),
        ),
    )(partials, tk_p, tv_p, rad)

    return (emd[0, 0], uni[0, 0])


# speedup vs baseline: 12.1548x; 1.0314x over previous
import functools

import jax
import jax.numpy as jnp
from jax import lax
from jax.experimental import pallas as pl
from jax.experimental.pallas import tpu as pltpu
from jax.experimental.pallas import tpu_sc as plsc

_B, _N = 8, 2048
_L = 16
_NC, _NS = 2, 16
_NW = _NC * _NS
_RPW = _B * _N // _NW
_WPB = _NW // _B
_GRP = _RPW // _L
_CHUNKS = _N // _L
_TR = 256
_NT = _N // _TR
_BIG = float(3.0e38)

_RADIUS = 0.07
_H = 0.03
_EPS = 1e-12


def _sc_body(pred_hbm, bpred_hbm, out_tk, out_tv, pco, bpc, qnb, tkb, tvb):
    wid = lax.axis_index("s") * _NC + lax.axis_index("c")
    b = wid // _WPB
    base = (wid % _WPB) * _RPW

    pltpu.sync_copy(pred_hbm.at[b], pco)
    pltpu.sync_copy(bpred_hbm.at[b], bpc)

    def prep(c, carry):
        s = pl.ds(c * _L, _L)
        qx = pco[s]
        qy = pco[pl.ds(_N + c * _L, _L)]
        qz = pco[pl.ds(2 * _N + c * _L, _L)]
        qnb[s] = qx * qx + qy * qy + qz * qz
        return carry

    lax.fori_loop(0, _CHUNKS, prep, 0)

    def grp16(g, carry):
        rbase = base + g * _L
        pxc = pco[pl.ds(rbase, _L)]
        pyc = pco[pl.ds(_N + rbase, _L)]
        pzc = pco[pl.ds(2 * _N + rbase, _L)]
        bxc = bpc[pl.ds(rbase, _L)]
        byc = bpc[pl.ds(_N + rbase, _L)]
        bzc = bpc[pl.ds(2 * _N + rbase, _L)]
        pnc = qnb[pl.ds(rbase, _L)]

        for pair in range(_L // 2):
            row = []
            for i in range(2):
                l = pair * 2 + i
                row.append((
                    jnp.broadcast_to(pxc[l], (_L,)),
                    jnp.broadcast_to(pyc[l], (_L,)),
                    jnp.broadcast_to(pzc[l], (_L,)),
                    jnp.broadcast_to(bxc[l], (_L,)),
                    jnp.broadcast_to(byc[l], (_L,)),
                    jnp.broadcast_to(bzc[l], (_L,)),
                    jnp.broadcast_to(pnc[l], (_L,)),
                ))

            init = (jnp.full((_L,), _BIG, jnp.float32),) * 20

            def chunk(c, st, row=row):
                st = list(st)
                s0 = pl.ds(c * _L, _L)
                s1 = pl.ds(_N + c * _L, _L)
                s2 = pl.ds(2 * _N + c * _L, _L)
                qx = pco[s0]
                qy = pco[s1]
                qz = pco[s2]
                bqx = bpc[s0]
                bqy = bpc[s1]
                bqz = bpc[s2]
                qn = qnb[s0]
                out = []
                for i in range(2):
                    px, py, pz, bx, by, bz, pn = row[i]
                    tk = list(st[i * 10:i * 10 + 5])
                    tv = list(st[i * 10 + 5:i * 10 + 10])
                    edot = bx * bqx + by * bqy + bz * bqz
                    ekey = (pn + qn) - 2.0 * edot
                    ex = px - qx
                    ey = py - qy
                    ez = pz - qz
                    ev = ex * ex + ey * ey + ez * ez
                    for lev in range(4):
                        cnd = ekey < tk[lev]
                        nk = jnp.minimum(tk[lev], ekey)
                        xk = jnp.maximum(tk[lev], ekey)
                        nv = jnp.where(cnd, ev, tv[lev])
                        xv = jnp.where(cnd, tv[lev], ev)
                        tk[lev] = nk
                        tv[lev] = nv
                        ekey = xk
                        ev = xv
                    cnd = ekey < tk[4]
                    tk[4] = jnp.minimum(tk[4], ekey)
                    tv[4] = jnp.where(cnd, ev, tv[4])
                    out += tk + tv
                return tuple(out)

            st = lax.fori_loop(0, _CHUNKS, chunk, init, unroll=2)
            for i in range(2):
                r = g * _L + pair * 2 + i
                for k in range(5):
                    tkb[pl.ds((r * 5 + k) * _L, _L)] = st[i * 10 + k]
                    tvb[pl.ds((r * 5 + k) * _L, _L)] = st[i * 10 + 5 + k]
        return carry

    lax.fori_loop(0, _GRP, grp16, 0)

    pltpu.sync_copy(tkb, out_tk.at[wid])
    pltpu.sync_copy(tvb, out_tv.at[wid])


_sc_knn = functools.partial(
    pl.kernel,
    out_type=(
        jax.ShapeDtypeStruct((_NW, _RPW * 5 * _L), jnp.float32),
        jax.ShapeDtypeStruct((_NW, _RPW * 5 * _L), jnp.float32),
    ),
    mesh=plsc.VectorSubcoreMesh(core_axis_name="c", subcore_axis_name="s",
                                num_cores=_NC, num_subcores=_NS),
    scratch_types=[
        pltpu.VMEM((3 * _N,), jnp.float32),
        pltpu.VMEM((3 * _N,), jnp.float32),
        pltpu.VMEM((_N,), jnp.float32),
        pltpu.VMEM((_RPW * 5 * _L,), jnp.float32),
        pltpu.VMEM((_RPW * 5 * _L,), jnp.float32),
    ],
)(_sc_body)


def _emd_body(p_ref, bp_ref, g_ref, bg_ref, out_ref):
    p = p_ref[0]
    bp = bp_ref[0]
    g = g_ref[0]
    bg = bg_ref[0]
    a2 = jnp.sum(p * p, axis=1, keepdims=True)
    b2 = jnp.sum(g * g, axis=0, keepdims=True)
    dot_k = jax.lax.dot_general(
        bp, bg, (((1,), (0,)), ((), ())),
        preferred_element_type=jnp.float32)
    key = (a2 + b2) - 2.0 * dot_k
    dot_v = jax.lax.dot_general(
        p, g, (((1,), (0,)), ((), ())),
        preferred_element_type=jnp.float32,
        precision=jax.lax.Precision.HIGHEST)
    val = (a2 + b2) - 2.0 * dot_v
    rowkey = jnp.min(key, axis=1, keepdims=True)
    rowval = jnp.min(jnp.where(key == rowkey, val, _BIG), axis=1,
                     keepdims=True)
    out_ref[pl.program_id(0), pl.program_id(1)] = jnp.sum(rowval)


def _reduce_body(pt_ref, tk_ref, tv_ref, rad_ref, emd_ref, uni_ref):
    tot = jnp.float32(0.0)
    for b in range(_B):
        pb = jnp.float32(0.0)
        for t in range(_NT):
            pb = pb + pt_ref[b, t]
        tot = tot + pb / rad_ref[b, 0]
    emd_ref[0, 0] = tot * (250.0 / (3.0 * _N * _B))

    tk = tk_ref[...]
    tv = tv_ref[...]
    k0 = jnp.min(tk, axis=1, keepdims=True)
    tk = jnp.where(tk == k0, _BIG, tk)
    acc = jnp.zeros((), jnp.float32)
    for k in range(4):
        kk = jnp.min(tk, axis=1, keepdims=True)
        sel = tk == kk
        vv = jnp.min(jnp.where(sel, tv, _BIG), axis=1, keepdims=True)
        v = jnp.maximum(vv, _EPS)
        dist = jnp.sqrt(v)
        w = jnp.exp(v * (-1.0 / (_H * _H)))
        acc = acc + jnp.sum((_RADIUS - dist) * w)
        if k < 3:
            tk = jnp.where(sel, _BIG, tk)
    uni_ref[0, 0] = acc / (_B * _N * 4.0)


def _round_bf16(x):
    u = lax.bitcast_convert_type(x, jnp.int32)
    r = (u + ((u >> 16) & 1) + 0x7FFF) & jnp.int32(-65536)
    return lax.bitcast_convert_type(r, jnp.float32)


def kernel(pred_fullpoint, gt_fullpoint, radius_data):
    pred_t = jnp.transpose(pred_fullpoint, (0, 2, 1))
    pred_f = pred_t.reshape(_B, 3 * _N)
    bpred_f = _round_bf16(pred_f)

    zpad_r = jnp.zeros((_B, _N, 5), jnp.float32)
    p_pad = jnp.concatenate([pred_fullpoint, zpad_r], axis=2)
    bp_pad = _round_bf16(p_pad)
    zpad_c = jnp.zeros((_B, 5, _N), jnp.float32)
    g_pad = jnp.concatenate(
        [jnp.transpose(gt_fullpoint, (0, 2, 1)), zpad_c], axis=1)
    bg_pad = _round_bf16(g_pad)

    tk, tv = _sc_knn(pred_f, bpred_f)

    partials = pl.pallas_call(
        _emd_body,
        grid=(_B, _NT),
        in_specs=[
            pl.BlockSpec((1, _TR, 8), lambda b, t: (b, t, 0)),
            pl.BlockSpec((1, _TR, 8), lambda b, t: (b, t, 0)),
            pl.BlockSpec((1, 8, _N), lambda b, t: (b, 0, 0)),
            pl.BlockSpec((1, 8, _N), lambda b, t: (b, 0, 0)),
        ],
        out_specs=pl.BlockSpec((_B, _NT), lambda b, t: (0, 0),
                               memory_space=pltpu.SMEM),
        out_shape=jax.ShapeDtypeStruct((_B, _NT), jnp.float32),
    )(p_pad, bp_pad, g_pad, bg_pad)
    pad = jnp.full((_B * _N, 128 - 5 * _L), _BIG, jnp.float32)
    tk_p = jnp.concatenate([tk.reshape(_B * _N, 5 * _L), pad], axis=1)
    tv_p = jnp.concatenate([tv.reshape(_B * _N, 5 * _L), pad], axis=1)
    rad = radius_data.reshape(_B, 1)

    emd, uni = pl.pallas_call(
        _reduce_body,
        in_specs=[
            pl.BlockSpec(memory_space=pltpu.SMEM),
            pl.BlockSpec((_B * _N, 128), lambda: (0, 0)),
            pl.BlockSpec((_B * _N, 128), lambda: (0, 0)),
            pl.BlockSpec(memory_space=pltpu.SMEM),
        ],
        out_shape=(
            jax.ShapeDtypeStruct((1, 1), jnp.float32),
            jax.ShapeDtypeStruct((1, 1), jnp.float32),
        ),
        out_specs=(
            pl.BlockSpec(memory_space=pltpu.SMEM),
            pl.BlockSpec(memory_space=pltpu.SMEM),
        ),
    )(partials, tk_p, tv_p, rad)

    return (emd[0, 0], uni[0, 0])
